# baseline (speedup 1.0000x reference)
import functools

import jax
import jax.numpy as jnp
from jax import lax
from jax.experimental import pallas as pl
from jax.experimental.pallas import tpu as pltpu
from jax.experimental.pallas import tpu_sc as plsc

N = 10000
E = 320000
DN = 128
DE = 16
G = 128

NC = 2
NS = 16
NW = NC * NS
CH = 64
NCH = 158
EPT = NCH * CH
E_PAD = NW * EPT
N_ACC = 10016
RPS = N_ACC // NS
ROWW = 144

_mesh = plsc.VectorSubcoreMesh(core_axis_name="c", subcore_axis_name="s")
_sc_params = pltpu.CompilerParams(use_tc_tiling_on_sc=False,
                                  needs_layout_passes=False)


def _lrelu(x):
    return jnp.maximum(x, 0.01 * x)


def _node_dense_body(nf_ref, wn_ref, bn_ref, we1n_ref, w2blk_ref, b2_ref,
                     hv_ref, u_ref, sn_ref):
    nf = nf_ref[...]
    hv = _lrelu(jnp.dot(nf, wn_ref[...], preferred_element_type=jnp.float32)
                + bn_ref[...][None, :])
    hv_ref[...] = hv
    u_ref[...] = jnp.dot(nf, we1n_ref[...], preferred_element_type=jnp.float32)
    sn_ref[...] = jnp.dot(hv, w2blk_ref[...],
                          preferred_element_type=jnp.float32) + b2_ref[...][None, :]


def _edge_v_body(ef_ref, we1e_ref, be1_ref, v_ref):
    v_ref[...] = jnp.dot(ef_ref[...], we1e_ref[...],
                         preferred_element_type=jnp.float32) + be1_ref[...][None, :]


def _max_reduce_body(mpart_ref, m_ref):
    m = jnp.max(mpart_ref[...], axis=0)
    mt = jnp.transpose(m, (1, 0))
    m_ref[...] = jnp.concatenate(
        [mt, jnp.zeros((mt.shape[0], 13), jnp.float32)], axis=1)


def _take16(x, idx):
    return x.at[idx].get(mode="promise_in_bounds")


def _sc_pass1(u_hbm, v_hbm, sn_hbm, ei_hbm, w2_hbm,
              l_hbm, t_hbm, mpart_hbm,
              ub0, ub1, vb0, vb1, eb0, eb1, ls0, ls1, snbuf, mt, w2b,
              semu0, semu1, semv0, semv1, semtw0, semtw1, semlw0, semlw1,
              semsn):
    c = lax.axis_index("c")
    s = lax.axis_index("s")
    w = c * NS + s
    base0 = w * EPT
    ubufs, vbufs, ebufs, lsts = (ub0, ub1), (vb0, vb1), (eb0, eb1), (ls0, ls1)
    semus, semvs = (semu0, semu1), (semv0, semv1)
    semtws, semlws = (semtw0, semtw1), (semlw0, semlw1)

    pltpu.sync_copy(w2_hbm, w2b)
    neg = jnp.full((16,), -1e30, jnp.float32)
    for i in range(3):
        def _init(j, _, i=i):
            mt[i, pl.ds(j * 16, 16)] = neg
            return 0
        lax.fori_loop(0, N // 16, _init, 0)

    lanes = lax.iota(jnp.int32, 16)

    def issue(b, ch):
        base = base0 + ch * CH
        pltpu.sync_copy(ei_hbm.at[:, pl.ds(base, CH)], ebufs[b])
        pltpu.async_copy(u_hbm.at[ebufs[b].at[0]], ubufs[b], semus[b])
        pltpu.async_copy(v_hbm.at[pl.ds(base, CH)], vbufs[b], semvs[b])

    def wait_in(b):
        pltpu.make_async_copy(u_hbm.at[pl.ds(0, CH)], ubufs[b],
                              semus[b]).wait()
        pltpu.make_async_copy(v_hbm.at[pl.ds(0, CH)], vbufs[b],
                              semvs[b]).wait()

    def drain_out(b):
        pltpu.make_async_copy(vbufs[b], t_hbm.at[pl.ds(0, CH)],
                              semtws[b]).wait()
        pltpu.make_async_copy(lsts[b], l_hbm.at[pl.ds(0, CH)],
                              semlws[b]).wait()

    def compute(b, ch):
        base = base0 + ch * CH
        ub, vb, ebf, ls = ubufs[b], vbufs[b], ebufs[b], lsts[b]
        pltpu.async_copy(sn_hbm.at[ebf.at[1]], snbuf, semsn)
        pltpu.make_async_copy(sn_hbm.at[pl.ds(0, CH)], snbuf, semsn).wait()

        def g_body(g, _, ub=ub, vb=vb, ebf=ebf, ls=ls):
            ev = lanes + g * 16
            dv = plsc.load_gather(ebf, [jnp.full((16,), 1, jnp.int32), ev])
            valid = (lanes + (base + g * 16)) < E
            for i in range(3):
                ihead = jnp.full((16,), i, jnp.int32)

                def f8_body(f8, acc, i=i, ev=ev, ub=ub, vb=vb):
                    w2vec = w2b[i * 8 + f8, :]
                    fb = i * G + f8 * 16
                    for k in range(16):
                        fcol = jnp.zeros((16,), jnp.int32) + (fb + k)
                        uvec = plsc.load_gather(ub, [ev, fcol])
                        vvec = plsc.load_gather(vb, [ev, fcol])
                        gv = uvec + vvec
                        t = jnp.maximum(gv, 0.01 * gv)
                        plsc.store_scatter(vb, [ev, fcol], t)
                        acc = acc + t * w2vec[k]
                    return acc

                acc = lax.fori_loop(0, 8, f8_body,
                                    jnp.zeros((16,), jnp.float32))
                snv = plsc.load_gather(snbuf, [ev, ihead])
                z = snv + acc
                lg = jnp.maximum(z, 0.01 * z)
                plsc.store_scatter(ls, [ev, ihead], lg)
                lg_eff = jnp.where(valid, lg, -1e30)
                sk, sv = plsc.sort_key_val(dv, lg_eff)
                for sh in (1, 2, 4, 8):
                    idx = jnp.maximum(lanes - sh, 0)
                    xk = _take16(sk, idx)
                    xv = _take16(sv, idx)
                    ok = (lanes >= sh) & (xk == sk)
                    sv = jnp.where(ok, jnp.maximum(sv, xv), sv)
                nxt = _take16(sk, jnp.minimum(lanes + 1, 15))
                last = (sk != nxt) | (lanes == 15)
                cur = plsc.load_gather(mt, [ihead, sk])
                plsc.store_scatter(mt, [ihead, sk], jnp.maximum(cur, sv),
                                   mask=last)
            return 0

        lax.fori_loop(0, CH // 16, g_body, 0)
        pltpu.async_copy(vb, t_hbm.at[pl.ds(base, CH)], semtws[b])
        pltpu.async_copy(ls, l_hbm.at[pl.ds(base, CH)], semlws[b])

    issue(0, 0)

    def pair(gp, _):
        for b in (0, 1):
            ch = gp * 2 + b
            nb = 1 - b
            wait_in(b)

            @pl.when(ch + 1 < NCH)
            def _(b=b, nb=nb, ch=ch):
                @pl.when(ch >= 1)
                def _():
                    drain_out(nb)
                issue(nb, ch + 1)

            compute(b, ch)
        return 0

    lax.fori_loop(0, NCH // 2, pair, 0)
    drain_out(0)
    drain_out(1)
    pltpu.sync_copy(mt, mpart_hbm.at[w])


def _sc_pass2(t_hbm, l_hbm, dst_hbm, m_hbm,
              pacc_hbm,
              acc, tb0, tb1, rw0, rw1, mb0, mb1, lb0, lb1, db0, db1,
              semt0, semt1, seml0, seml1, semm0, semm1, sems0, sems1):
    c = lax.axis_index("c")
    s = lax.axis_index("s")
    w = c * NS + s
    base0 = w * EPT
    row0 = s * RPS
    tbufs, rowss, mbufs = (tb0, tb1), (rw0, rw1), (mb0, mb1)
    lbufs, dbufs = (lb0, lb1), (db0, db1)
    semts, semls = (semt0, semt1), (seml0, seml1)
    semms, semss = (semm0, semm1), (sems0, sems1)

    zero16 = jnp.zeros((16,), jnp.float32)
    lanes = lax.iota(jnp.int32, 16)
    colex = jnp.full((16,), G, jnp.int32)
    TAIL = RPS - (RPS // CH) * CH

    for i in range(3):
        def _zr(e, _):
            for k in range(ROWW // 16):
                rw0[e, pl.ds(k * 16, 16)] = zero16
                rw1[e, pl.ds(k * 16, 16)] = zero16
            return 0
        lax.fori_loop(0, CH, _zr, 0)
        for j in range(RPS // CH):
            pltpu.sync_copy(rw0, acc.at[pl.ds(row0 + j * CH, CH)])
        pltpu.sync_copy(rw0.at[pl.ds(0, TAIL)],
                        acc.at[pl.ds(row0 + (RPS // CH) * CH, TAIL)])
        plsc.subcore_barrier()
        ihead = jnp.full((16,), i, jnp.int32)

        def issue(b, ch, i=i):
            base = base0 + ch * CH
            pltpu.sync_copy(dst_hbm.at[pl.ds(base, CH)], dbufs[b])
            pltpu.async_copy(t_hbm.at[pl.ds(base, CH), pl.ds(i * G, G)],
                             tbufs[b], semts[b])
            pltpu.async_copy(l_hbm.at[pl.ds(base, CH)], lbufs[b], semls[b])
            pltpu.async_copy(m_hbm.at[dbufs[b]], mbufs[b], semms[b])

        def wait_in(b, i=i):
            pltpu.make_async_copy(t_hbm.at[pl.ds(0, CH), pl.ds(i * G, G)],
                                  tbufs[b], semts[b]).wait()
            pltpu.make_async_copy(l_hbm.at[pl.ds(0, CH)], lbufs[b],
                                  semls[b]).wait()
            pltpu.make_async_copy(m_hbm.at[pl.ds(0, CH)], mbufs[b],
                                  semms[b]).wait()

        def drain_sc(b):
            pltpu.make_async_copy(rowss[b], acc.at[pl.ds(0, CH)],
                                  semss[b]).wait()

        def compute(b, ch, ihead=ihead):
            base = base0 + ch * CH
            tb, rows, mb, lb = tbufs[b], rowss[b], mbufs[b], lbufs[b]

            def g_body(g, _, tb=tb, rows=rows, mb=mb, lb=lb):
                ev = lanes + g * 16
                mv = plsc.load_gather(mb, [ev, ihead])
                lv = plsc.load_gather(lb, [ev, ihead])
                ex = jnp.exp(lv - mv)
                mask = (lanes + (base + g * 16)) < E
                ex = jnp.where(mask, ex, 0.0)
                plsc.store_scatter(rows, [ev, colex], ex)

                def f8_body(f8, _, ev=ev, ex=ex, tb=tb, rows=rows):
                    for k in range(16):
                        fcol = jnp.zeros((16,), jnp.int32) + (f8 * 16 + k)
                        tv = plsc.load_gather(tb, [ev, fcol])
                        plsc.store_scatter(rows, [ev, fcol], tv * ex)
                    return 0
                lax.fori_loop(0, 8, f8_body, 0)
                return 0

            lax.fori_loop(0, CH // 16, g_body, 0)
            pltpu.async_copy(rows, acc.at[dbufs[b]], semss[b], add=True)

        issue(0, 0)

        def pair(gp, _):
            for b in (0, 1):
                ch = gp * 2 + b
                nb = 1 - b
                wait_in(b)

                @pl.when(ch + 1 < NCH)
                def _(b=b, nb=nb, ch=ch):
                    @pl.when(ch >= 1)
                    def _():
                        drain_sc(nb)
                    issue(nb, ch + 1)

                compute(b, ch)
            return 0

        lax.fori_loop(0, NCH // 2, pair, 0)
        drain_sc(0)
        drain_sc(1)
        plsc.subcore_barrier()

        @pl.when(s < NS - 1)
        def _():
            pltpu.sync_copy(acc.at[pl.ds(row0, RPS)],
                            pacc_hbm.at[c, i, pl.ds(row0, RPS)])

        @pl.when(s == NS - 1)
        def _():
            pltpu.sync_copy(acc.at[pl.ds(row0, N - (NS - 1) * RPS)],
                            pacc_hbm.at[c, i, pl.ds(row0, N - (NS - 1) * RPS)])
        plsc.subcore_barrier()


def _epilogue_body(pacc_ref, hv_ref,
                   wet_ref, bet_ref, wmca_ref, bmca_ref, wmcn_ref, bmcn_ref,
                   wih_ref, bih_ref, whh_ref, bhh_ref,
                   out_ref):
    pacc = pacc_ref[...]
    hv = hv_ref[...]
    ctx = []
    for i in range(3):
        p = pacc[0, i] + pacc[1, i]
        pi = p[:, :G]
        si = p[:, G]
        re = jnp.where(si > 0, 1.0 / jnp.where(si > 0, si, 1.0), 0.0)
        a = pi * re[:, None]
        ci = jnp.dot(a, wet_ref[...][i], preferred_element_type=jnp.float32)
        ci = ci + jnp.where(si > 0, 1.0, 0.0)[:, None] * bet_ref[...][i][None, :]
        ctx.append(jnp.where(ci > 0, ci, jnp.exp(jnp.minimum(ci, 0.0)) - 1.0))
    context = jnp.dot(jnp.concatenate(ctx, axis=1), wmca_ref[...],
                      preferred_element_type=jnp.float32) + bmca_ref[...][None, :]
    hnode = jnp.dot(hv, wmcn_ref[...],
                    preferred_element_type=jnp.float32) + bmcn_ref[...][None, :]
    gi = jnp.dot(context, wih_ref[...],
                 preferred_element_type=jnp.float32) + bih_ref[...][None, :]
    gh = jnp.dot(hnode, whh_ref[...],
                 preferred_element_type=jnp.float32) + bhh_ref[...][None, :]
    i_r, i_z, i_n = gi[:, :G], gi[:, G:2 * G], gi[:, 2 * G:]
    h_r, h_z, h_n = gh[:, :G], gh[:, G:2 * G], gh[:, 2 * G:]
    r = jax.nn.sigmoid(i_r + h_r)
    z = jax.nn.sigmoid(i_z + h_z)
    cand = jnp.tanh(i_n + r * h_n)
    h_new = (1.0 - z) * cand + z * hnode
    out_ref[...] = jnp.maximum(h_new, 0.0)


def kernel(node_feats, edge_feats, params, edge_index):
    p = params
    f32 = jnp.float32

    wn_cat = jnp.concatenate([p['Wn%d' % i] for i in (1, 2, 3)], axis=1)
    bn_cat = jnp.concatenate([p['bn%d' % i] for i in (1, 2, 3)], axis=0)
    we1n_cat = jnp.concatenate([p['We1_%d' % i][:DN] for i in (1, 2, 3)], axis=1)
    we1e_cat = jnp.concatenate([p['We1_%d' % i][DN:] for i in (1, 2, 3)], axis=1)
    be1_cat = jnp.concatenate([p['be1_%d' % i] for i in (1, 2, 3)], axis=0)
    w2blk = jnp.zeros((3 * G, 16), f32)
    for i in (1, 2, 3):
        w2blk = w2blk.at[(i - 1) * G:i * G, i - 1].set(p['We2_%d' % i][:G, 0])
    b2 = jnp.zeros((16,), f32)
    for i in (1, 2, 3):
        b2 = b2.at[i - 1].set(p['be2_%d' % i][0])
    w2b = jnp.stack([p['We2_%d' % i][G:, 0] for i in (1, 2, 3)], axis=0)
    wet = jnp.stack([p['Wet%d' % i] for i in (1, 2, 3)], axis=0)
    bet = jnp.stack([p['bet%d' % i] for i in (1, 2, 3)], axis=0)

    ei_pad = jnp.pad(edge_index.astype(jnp.int32), ((0, 0), (0, E_PAD - E)))
    dst = ei_pad[1]
    ef_pad = jnp.pad(edge_feats, ((0, E_PAD - E), (0, 0)))

    hv_cat, u_cat, sn3 = pl.pallas_call(
        _node_dense_body,
        out_shape=[jax.ShapeDtypeStruct((N, 3 * G), f32),
                   jax.ShapeDtypeStruct((N, 3 * G), f32),
                   jax.ShapeDtypeStruct((N, 16), f32)],
    )(node_feats, wn_cat, bn_cat, we1n_cat, w2blk, b2)

    EB = 2528
    v_cat = pl.pallas_call(
        _edge_v_body,
        grid=(E_PAD // EB,),
        in_specs=[pl.BlockSpec((EB, DE), lambda i: (i, 0)),
                  pl.BlockSpec((DE, 3 * G), lambda i: (0, 0)),
                  pl.BlockSpec((3 * G,), lambda i: (0,))],
        out_specs=pl.BlockSpec((EB, 3 * G), lambda i: (i, 0)),
        out_shape=jax.ShapeDtypeStruct((E_PAD, 3 * G), f32),
    )(ef_pad, we1e_cat, be1_cat)

    pass1 = pl.kernel(
        _sc_pass1,
        out_type=[jax.ShapeDtypeStruct((E_PAD, 4), f32),
                  jax.ShapeDtypeStruct((E_PAD, 3 * G), f32),
                  jax.ShapeDtypeStruct((NW, 3, N), f32)],
        mesh=_mesh,
        scratch_types=[
            pltpu.VMEM((CH, 3 * G), f32),
            pltpu.VMEM((CH, 3 * G), f32),
            pltpu.VMEM((CH, 3 * G), f32),
            pltpu.VMEM((CH, 3 * G), f32),
            pltpu.VMEM((2, CH), jnp.int32),
            pltpu.VMEM((2, CH), jnp.int32),
            pltpu.VMEM((CH, 4), f32),
            pltpu.VMEM((CH, 4), f32),
            pltpu.VMEM((CH, 16), f32),
            pltpu.VMEM((3, N), f32),
            pltpu.VMEM((24, 16), f32),
        ] + [pltpu.SemaphoreType.DMA] * 9,
        compiler_params=_sc_params,
    )
    logits, he1t, mpart = pass1(u_cat, v_cat, sn3, ei_pad,
                                w2b.reshape(24, 16))

    m3 = pl.pallas_call(
        _max_reduce_body,
        out_shape=jax.ShapeDtypeStruct((N, 16), f32),
    )(mpart)

    pass2 = pl.kernel(
        _sc_pass2,
        out_type=jax.ShapeDtypeStruct((NC, 3, N, ROWW), f32),
        mesh=_mesh,
        scratch_types=[
            pltpu.VMEM_SHARED((N_ACC, ROWW), f32),
            pltpu.VMEM((CH, G), f32),
            pltpu.VMEM((CH, G), f32),
            pltpu.VMEM((CH, ROWW), f32),
            pltpu.VMEM((CH, ROWW), f32),
            pltpu.VMEM((CH, 16), f32),
            pltpu.VMEM((CH, 16), f32),
            pltpu.VMEM((CH, 4), f32),
            pltpu.VMEM((CH, 4), f32),
            pltpu.VMEM((CH,), jnp.int32),
            pltpu.VMEM((CH,), jnp.int32),
        ] + [pltpu.SemaphoreType.DMA] * 8,
        compiler_params=_sc_params,
    )
    pacc = pass2(he1t, logits, dst, m3)

    NB = 2000
    out = pl.pallas_call(
        _epilogue_body,
        grid=(N // NB,),
        in_specs=[pl.BlockSpec((NC, 3, NB, ROWW), lambda k: (0, 0, k, 0)),
                  pl.BlockSpec((NB, 3 * G), lambda k: (k, 0)),
                  pl.BlockSpec((3, G, G), lambda k: (0, 0, 0)),
                  pl.BlockSpec((3, G), lambda k: (0, 0)),
                  pl.BlockSpec((3 * G, G), lambda k: (0, 0)),
                  pl.BlockSpec((G,), lambda k: (0,)),
                  pl.BlockSpec((3 * G, G), lambda k: (0, 0)),
                  pl.BlockSpec((G,), lambda k: (0,)),
                  pl.BlockSpec((G, 3 * G), lambda k: (0, 0)),
                  pl.BlockSpec((3 * G,), lambda k: (0,)),
                  pl.BlockSpec((G, 3 * G), lambda k: (0, 0)),
                  pl.BlockSpec((3 * G,), lambda k: (0,))],
        out_specs=pl.BlockSpec((NB, G), lambda k: (k, 0)),
        out_shape=jax.ShapeDtypeStruct((N, G), f32),
    )(pacc, hv_cat, wet, bet, p['Wmca'], p['bmca'], p['Wmcn'], p['bmcn'],
      p['W_ih'], p['b_ih'], p['W_hh'], p['b_hh'])
    return out


# speedup vs baseline: 2.5306x; 2.5306x over previous
import functools

import jax
import jax.numpy as jnp
from jax import lax
from jax.experimental import pallas as pl
from jax.experimental.pallas import tpu as pltpu
from jax.experimental.pallas import tpu_sc as plsc

N = 10000
E = 320000
DN = 128
DE = 16
G = 128

NC = 2
NS = 16
NW = NC * NS
CH = 64
NCH = 158
EPT = NCH * CH
E_PAD = NW * EPT
N_ACC = 10016
RPS = N_ACC // NS
ROWW = 144

_mesh = plsc.VectorSubcoreMesh(core_axis_name="c", subcore_axis_name="s")
_sc_params = pltpu.CompilerParams(use_tc_tiling_on_sc=False,
                                  needs_layout_passes=False)


def _lrelu(x):
    return jnp.maximum(x, 0.01 * x)


def _node_dense_body(nf_ref, wn_ref, bn_ref, we1n_ref, w2blk_ref, b2_ref,
                     hv_ref, u_ref, sn_ref):
    nf = nf_ref[...]
    hv = _lrelu(jnp.dot(nf, wn_ref[...], preferred_element_type=jnp.float32)
                + bn_ref[...][None, :])
    hv_ref[...] = hv
    u_ref[...] = jnp.dot(nf, we1n_ref[...], preferred_element_type=jnp.float32)
    sn_ref[...] = jnp.dot(hv, w2blk_ref[...],
                          preferred_element_type=jnp.float32) + b2_ref[...][None, :]


def _edge_v_body(ef_ref, we1e_ref, be1_ref, v_ref):
    v_ref[...] = jnp.dot(ef_ref[...], we1e_ref[...],
                         preferred_element_type=jnp.float32) + be1_ref[...][None, :]


def _max_reduce_body(mpart_ref, m_ref):
    m = jnp.max(mpart_ref[...], axis=0)
    mt = jnp.transpose(m, (1, 0))
    m_ref[...] = jnp.concatenate(
        [mt, jnp.zeros((mt.shape[0], 13), jnp.float32)], axis=1)


def _take16(x, idx):
    return x.at[idx].get(mode="promise_in_bounds")


def _sc_pass1(u_hbm, v_hbm, sn_hbm, ei_hbm, w2_hbm,
              l_hbm, t_hbm,
              ub0, ub1, vb0, vb1, eb0, eb1, ls0, ls1, snbuf, ts, dotb, w2b,
              semu0, semu1, semv0, semv1, semtw, semlw0, semlw1,
              semsn):
    c = lax.axis_index("c")
    s = lax.axis_index("s")
    w = c * NS + s
    base0 = w * EPT
    ubufs, vbufs, ebufs, lsts = (ub0, ub1), (vb0, vb1), (eb0, eb1), (ls0, ls1)
    semus, semvs = (semu0, semu1), (semv0, semv1)
    semlws = (semlw0, semlw1)

    pltpu.sync_copy(w2_hbm, w2b)
    w2v = [[w2b[i * 8 + j, :] for j in range(8)] for i in range(3)]

    lanes = lax.iota(jnp.int32, 16)

    def issue(b, ch):
        base = base0 + ch * CH
        pltpu.sync_copy(ei_hbm.at[:, pl.ds(base, CH)], ebufs[b])
        pltpu.async_copy(u_hbm.at[ebufs[b].at[0]], ubufs[b], semus[b])
        pltpu.async_copy(v_hbm.at[pl.ds(base, CH)], vbufs[b], semvs[b])

    def wait_in(b):
        pltpu.make_async_copy(u_hbm.at[pl.ds(0, CH)], ubufs[b],
                              semus[b]).wait()
        pltpu.make_async_copy(v_hbm.at[pl.ds(0, CH)], vbufs[b],
                              semvs[b]).wait()

    def drain_out(b):
        pltpu.make_async_copy(lsts[b], l_hbm.at[pl.ds(0, CH)],
                              semlws[b]).wait()

    def drain_tw(ch):
        @pl.when(ch >= 1)
        def _():
            for i in range(3):
                pltpu.make_async_copy(ts.at[i], t_hbm.at[i, pl.ds(0, CH)],
                                      semtw).wait()

    def compute(b, ch):
        base = base0 + ch * CH
        ub, vb, ebf, ls = ubufs[b], vbufs[b], ebufs[b], lsts[b]
        pltpu.async_copy(sn_hbm.at[ebf.at[1]], snbuf, semsn)
        pltpu.make_async_copy(sn_hbm.at[pl.ds(0, CH)], snbuf, semsn).wait()
        drain_tw(ch)

        def g_body(g, _, ub=ub, vb=vb, ls=ls):
            ev = lanes + g * 16
            for i in range(3):
                ihead = jnp.full((16,), i, jnp.int32)

                def e_body(e16, _, g=g, i=i, ub=ub, vb=vb):
                    e = g * 16 + e16
                    acc = None
                    for j in range(8):
                        sl = pl.ds(i * G + j * 16, 16)
                        gg = ub[e, sl] + vb[e, sl]
                        t = jnp.maximum(gg, 0.01 * gg)
                        ts[i, e, pl.ds(j * 16, 16)] = t
                        pj = t * w2v[i][j]
                        acc = pj if acc is None else acc + pj
                    dotb[e16, :] = acc
                    return 0

                lax.fori_loop(0, 16, e_body, 0)
                se = None
                for j in range(16):
                    cj = plsc.load_gather(
                        dotb, [lanes, jnp.full((16,), j, jnp.int32)])
                    se = cj if se is None else se + cj
                snv = plsc.load_gather(snbuf, [ev, ihead])
                z = snv + se
                lg = jnp.maximum(z, 0.01 * z)
                plsc.store_scatter(ls, [ev, ihead], lg)
            return 0

        lax.fori_loop(0, CH // 16, g_body, 0)
        for i in range(3):
            pltpu.async_copy(ts.at[i], t_hbm.at[i, pl.ds(base, CH)], semtw)
        pltpu.async_copy(ls, l_hbm.at[pl.ds(base, CH)], semlws[b])

    issue(0, 0)

    def pair(gp, _):
        for b in (0, 1):
            ch = gp * 2 + b
            nb = 1 - b
            wait_in(b)

            @pl.when(ch + 1 < NCH)
            def _(b=b, nb=nb, ch=ch):
                @pl.when(ch >= 1)
                def _():
                    drain_out(nb)
                issue(nb, ch + 1)

            compute(b, ch)
        return 0

    lax.fori_loop(0, NCH // 2, pair, 0)
    drain_out(0)
    drain_out(1)
    for i in range(3):
        pltpu.make_async_copy(ts.at[i], t_hbm.at[i, pl.ds(0, CH)],
                              semtw).wait()


def _sc_segmax(l_hbm, dst_hbm, mpart_hbm,
               mt, lb0, lb1, db0, db1, seml0, seml1, semd0, semd1):
    c = lax.axis_index("c")
    s = lax.axis_index("s")
    w = c * NS + s
    base0 = w * EPT
    lbufs, dbufs = (lb0, lb1), (db0, db1)
    semls, semds = (seml0, seml1), (semd0, semd1)

    neg = jnp.full((16,), -1e30, jnp.float32)
    for i in range(3):
        def _init(j, _, i=i):
            mt[i, pl.ds(j * 16, 16)] = neg
            return 0
        lax.fori_loop(0, N // 16, _init, 0)

    lanes = lax.iota(jnp.int32, 16)

    def issue(b, ch):
        base = base0 + ch * CH
        pltpu.async_copy(l_hbm.at[pl.ds(base, CH)], lbufs[b], semls[b])
        pltpu.async_copy(dst_hbm.at[pl.ds(base, CH)], dbufs[b], semds[b])

    def wait_in(b):
        pltpu.make_async_copy(l_hbm.at[pl.ds(0, CH)], lbufs[b],
                              semls[b]).wait()
        pltpu.make_async_copy(dst_hbm.at[pl.ds(0, CH)], dbufs[b],
                              semds[b]).wait()

    def compute(b, ch):
        base = base0 + ch * CH
        lb, db = lbufs[b], dbufs[b]

        def g_body(g, _, lb=lb, db=db):
            ev = lanes + g * 16
            dv = db[pl.ds(g * 16, 16)]
            valid = (lanes + (base + g * 16)) < E
            for i in range(3):
                ihead = jnp.full((16,), i, jnp.int32)
                lv = plsc.load_gather(lb, [ev, ihead])
                lg_eff = jnp.where(valid, lv, -1e30)
                sk, sv = plsc.sort_key_val(dv, lg_eff)
                for sh in (1, 2, 4, 8):
                    idx = jnp.maximum(lanes - sh, 0)
                    xk = _take16(sk, idx)
                    xv = _take16(sv, idx)
                    ok = (lanes >= sh) & (xk == sk)
                    sv = jnp.where(ok, jnp.maximum(sv, xv), sv)
                nxt = _take16(sk, jnp.minimum(lanes + 1, 15))
                last = (sk != nxt) | (lanes == 15)
                cur = plsc.load_gather(mt, [ihead, sk])
                plsc.store_scatter(mt, [ihead, sk], jnp.maximum(cur, sv),
                                   mask=last)
            return 0

        lax.fori_loop(0, CH // 16, g_body, 0)

    issue(0, 0)

    def pair(gp, _):
        for b in (0, 1):
            ch = gp * 2 + b
            nb = 1 - b
            wait_in(b)

            @pl.when(ch + 1 < NCH)
            def _(b=b, nb=nb, ch=ch):
                issue(nb, ch + 1)

            compute(b, ch)
        return 0

    lax.fori_loop(0, NCH // 2, pair, 0)
    pltpu.sync_copy(mt, mpart_hbm.at[w])


def _sc_pass2(t_hbm, l_hbm, dst_hbm, m_hbm,
              pacc_hbm,
              acc, tb0, tb1, rw0, rw1, mb0, mb1, lb0, lb1, db0, db1, exb,
              semt0, semt1, seml0, seml1, semm0, semm1, sems0, sems1):
    c = lax.axis_index("c")
    s = lax.axis_index("s")
    w = c * NS + s
    base0 = w * EPT
    row0 = s * RPS
    tbufs, rowss, mbufs = (tb0, tb1), (rw0, rw1), (mb0, mb1)
    lbufs, dbufs = (lb0, lb1), (db0, db1)
    semts, semls = (semt0, semt1), (seml0, seml1)
    semms, semss = (semm0, semm1), (sems0, sems1)

    zero16 = jnp.zeros((16,), jnp.float32)
    lanes = lax.iota(jnp.int32, 16)
    colex = jnp.full((16,), G, jnp.int32)
    TAIL = RPS - (RPS // CH) * CH

    for i in range(3):
        def _zr(e, _):
            for k in range(ROWW // 16):
                rw0[e, pl.ds(k * 16, 16)] = zero16
                rw1[e, pl.ds(k * 16, 16)] = zero16
            return 0
        lax.fori_loop(0, CH, _zr, 0)
        for j in range(RPS // CH):
            pltpu.sync_copy(rw0, acc.at[pl.ds(row0 + j * CH, CH)])
        pltpu.sync_copy(rw0.at[pl.ds(0, TAIL)],
                        acc.at[pl.ds(row0 + (RPS // CH) * CH, TAIL)])
        plsc.subcore_barrier()
        ihead = jnp.full((16,), i, jnp.int32)

        def issue(b, ch, i=i):
            base = base0 + ch * CH
            pltpu.sync_copy(dst_hbm.at[pl.ds(base, CH)], dbufs[b])
            pltpu.async_copy(t_hbm.at[i, pl.ds(base, CH)], tbufs[b],
                             semts[b])
            pltpu.async_copy(l_hbm.at[pl.ds(base, CH)], lbufs[b], semls[b])
            pltpu.async_copy(m_hbm.at[dbufs[b]], mbufs[b], semms[b])

        def wait_in(b, i=i):
            pltpu.make_async_copy(t_hbm.at[i, pl.ds(0, CH)], tbufs[b],
                                  semts[b]).wait()
            pltpu.make_async_copy(l_hbm.at[pl.ds(0, CH)], lbufs[b],
                                  semls[b]).wait()
            pltpu.make_async_copy(m_hbm.at[pl.ds(0, CH)], mbufs[b],
                                  semms[b]).wait()

        def drain_sc(b):
            pltpu.make_async_copy(rowss[b], acc.at[pl.ds(0, CH)],
                                  semss[b]).wait()

        def compute(b, ch, ihead=ihead):
            base = base0 + ch * CH
            tb, rows, mb, lb = tbufs[b], rowss[b], mbufs[b], lbufs[b]

            def g_body(g, _, rows=rows, mb=mb, lb=lb):
                ev = lanes + g * 16
                mv = plsc.load_gather(mb, [ev, ihead])
                lv = plsc.load_gather(lb, [ev, ihead])
                ex = jnp.exp(lv - mv)
                mask = (lanes + (base + g * 16)) < E
                ex = jnp.where(mask, ex, 0.0)
                plsc.store_scatter(rows, [ev, colex], ex)
                exb[pl.ds(g * 16, 16)] = ex
                return 0

            lax.fori_loop(0, CH // 16, g_body, 0)

            def e_body(e, _, tb=tb, rows=rows):
                exv = plsc.load_gather(exb, [jnp.zeros((16,), jnp.int32) + e])
                for j in range(8):
                    sl = pl.ds(j * 16, 16)
                    rows[e, sl] = tb[e, sl] * exv
                return 0

            lax.fori_loop(0, CH, e_body, 0)
            pltpu.async_copy(rows, acc.at[dbufs[b]], semss[b], add=True)

        issue(0, 0)

        def pair(gp, _):
            for b in (0, 1):
                ch = gp * 2 + b
                nb = 1 - b
                wait_in(b)

                @pl.when(ch + 1 < NCH)
                def _(b=b, nb=nb, ch=ch):
                    @pl.when(ch >= 1)
                    def _():
                        drain_sc(nb)
                    issue(nb, ch + 1)

                compute(b, ch)
            return 0

        lax.fori_loop(0, NCH // 2, pair, 0)
        drain_sc(0)
        drain_sc(1)
        plsc.subcore_barrier()

        @pl.when(s < NS - 1)
        def _():
            pltpu.sync_copy(acc.at[pl.ds(row0, RPS)],
                            pacc_hbm.at[c, i, pl.ds(row0, RPS)])

        @pl.when(s == NS - 1)
        def _():
            pltpu.sync_copy(acc.at[pl.ds(row0, N - (NS - 1) * RPS)],
                            pacc_hbm.at[c, i, pl.ds(row0, N - (NS - 1) * RPS)])
        plsc.subcore_barrier()


def _epilogue_body(pacc_ref, hv_ref,
                   wet_ref, bet_ref, wmca_ref, bmca_ref, wmcn_ref, bmcn_ref,
                   wih_ref, bih_ref, whh_ref, bhh_ref,
                   out_ref):
    pacc = pacc_ref[...]
    hv = hv_ref[...]
    ctx = []
    for i in range(3):
        p = pacc[0, i] + pacc[1, i]
        pi = p[:, :G]
        si = p[:, G]
        re = jnp.where(si > 0, 1.0 / jnp.where(si > 0, si, 1.0), 0.0)
        a = pi * re[:, None]
        ci = jnp.dot(a, wet_ref[...][i], preferred_element_type=jnp.float32)
        ci = ci + jnp.where(si > 0, 1.0, 0.0)[:, None] * bet_ref[...][i][None, :]
        ctx.append(jnp.where(ci > 0, ci, jnp.exp(jnp.minimum(ci, 0.0)) - 1.0))
    context = jnp.dot(jnp.concatenate(ctx, axis=1), wmca_ref[...],
                      preferred_element_type=jnp.float32) + bmca_ref[...][None, :]
    hnode = jnp.dot(hv, wmcn_ref[...],
                    preferred_element_type=jnp.float32) + bmcn_ref[...][None, :]
    gi = jnp.dot(context, wih_ref[...],
                 preferred_element_type=jnp.float32) + bih_ref[...][None, :]
    gh = jnp.dot(hnode, whh_ref[...],
                 preferred_element_type=jnp.float32) + bhh_ref[...][None, :]
    i_r, i_z, i_n = gi[:, :G], gi[:, G:2 * G], gi[:, 2 * G:]
    h_r, h_z, h_n = gh[:, :G], gh[:, G:2 * G], gh[:, 2 * G:]
    r = jax.nn.sigmoid(i_r + h_r)
    z = jax.nn.sigmoid(i_z + h_z)
    cand = jnp.tanh(i_n + r * h_n)
    h_new = (1.0 - z) * cand + z * hnode
    out_ref[...] = jnp.maximum(h_new, 0.0)


def kernel(node_feats, edge_feats, params, edge_index):
    p = params
    f32 = jnp.float32

    wn_cat = jnp.concatenate([p['Wn%d' % i] for i in (1, 2, 3)], axis=1)
    bn_cat = jnp.concatenate([p['bn%d' % i] for i in (1, 2, 3)], axis=0)
    we1n_cat = jnp.concatenate([p['We1_%d' % i][:DN] for i in (1, 2, 3)], axis=1)
    we1e_cat = jnp.concatenate([p['We1_%d' % i][DN:] for i in (1, 2, 3)], axis=1)
    be1_cat = jnp.concatenate([p['be1_%d' % i] for i in (1, 2, 3)], axis=0)
    w2blk = jnp.zeros((3 * G, 16), f32)
    for i in (1, 2, 3):
        w2blk = w2blk.at[(i - 1) * G:i * G, i - 1].set(p['We2_%d' % i][:G, 0])
    b2 = jnp.zeros((16,), f32)
    for i in (1, 2, 3):
        b2 = b2.at[i - 1].set(p['be2_%d' % i][0])
    w2b = jnp.stack([p['We2_%d' % i][G:, 0] for i in (1, 2, 3)], axis=0)
    wet = jnp.stack([p['Wet%d' % i] for i in (1, 2, 3)], axis=0)
    bet = jnp.stack([p['bet%d' % i] for i in (1, 2, 3)], axis=0)

    ei_pad = jnp.pad(edge_index.astype(jnp.int32), ((0, 0), (0, E_PAD - E)))
    dst = ei_pad[1]
    ef_pad = jnp.pad(edge_feats, ((0, E_PAD - E), (0, 0)))

    hv_cat, u_cat, sn3 = pl.pallas_call(
        _node_dense_body,
        out_shape=[jax.ShapeDtypeStruct((N, 3 * G), f32),
                   jax.ShapeDtypeStruct((N, 3 * G), f32),
                   jax.ShapeDtypeStruct((N, 16), f32)],
    )(node_feats, wn_cat, bn_cat, we1n_cat, w2blk, b2)

    EB = 2528
    v_cat = pl.pallas_call(
        _edge_v_body,
        grid=(E_PAD // EB,),
        in_specs=[pl.BlockSpec((EB, DE), lambda i: (i, 0)),
                  pl.BlockSpec((DE, 3 * G), lambda i: (0, 0)),
                  pl.BlockSpec((3 * G,), lambda i: (0,))],
        out_specs=pl.BlockSpec((EB, 3 * G), lambda i: (i, 0)),
        out_shape=jax.ShapeDtypeStruct((E_PAD, 3 * G), f32),
    )(ef_pad, we1e_cat, be1_cat)

    pass1 = pl.kernel(
        _sc_pass1,
        out_type=[jax.ShapeDtypeStruct((E_PAD, 4), f32),
                  jax.ShapeDtypeStruct((3, E_PAD, G), f32)],
        mesh=_mesh,
        scratch_types=[
            pltpu.VMEM((CH, 3 * G), f32),
            pltpu.VMEM((CH, 3 * G), f32),
            pltpu.VMEM((CH, 3 * G), f32),
            pltpu.VMEM((CH, 3 * G), f32),
            pltpu.VMEM((2, CH), jnp.int32),
            pltpu.VMEM((2, CH), jnp.int32),
            pltpu.VMEM((CH, 4), f32),
            pltpu.VMEM((CH, 4), f32),
            pltpu.VMEM((CH, 16), f32),
            pltpu.VMEM((3, CH, G), f32),
            pltpu.VMEM((16, 16), f32),
            pltpu.VMEM((24, 16), f32),
        ] + [pltpu.SemaphoreType.DMA] * 8,
        compiler_params=_sc_params,
    )
    logits, he1t = pass1(u_cat, v_cat, sn3, ei_pad, w2b.reshape(24, 16))

    segmax = pl.kernel(
        _sc_segmax,
        out_type=jax.ShapeDtypeStruct((NW, 3, N), f32),
        mesh=_mesh,
        scratch_types=[
            pltpu.VMEM((3, N), f32),
            pltpu.VMEM((CH, 4), f32),
            pltpu.VMEM((CH, 4), f32),
            pltpu.VMEM((CH,), jnp.int32),
            pltpu.VMEM((CH,), jnp.int32),
        ] + [pltpu.SemaphoreType.DMA] * 4,
        compiler_params=_sc_params,
    )
    mpart = segmax(logits, dst)

    m3 = pl.pallas_call(
        _max_reduce_body,
        out_shape=jax.ShapeDtypeStruct((N, 16), f32),
    )(mpart)

    pass2 = pl.kernel(
        _sc_pass2,
        out_type=jax.ShapeDtypeStruct((NC, 3, N, ROWW), f32),
        mesh=_mesh,
        scratch_types=[
            pltpu.VMEM_SHARED((N_ACC, ROWW), f32),
            pltpu.VMEM((CH, G), f32),
            pltpu.VMEM((CH, G), f32),
            pltpu.VMEM((CH, ROWW), f32),
            pltpu.VMEM((CH, ROWW), f32),
            pltpu.VMEM((CH, 16), f32),
            pltpu.VMEM((CH, 16), f32),
            pltpu.VMEM((CH, 4), f32),
            pltpu.VMEM((CH, 4), f32),
            pltpu.VMEM((CH,), jnp.int32),
            pltpu.VMEM((CH,), jnp.int32),
            pltpu.VMEM((CH,), f32),
        ] + [pltpu.SemaphoreType.DMA] * 8,
        compiler_params=_sc_params,
    )
    pacc = pass2(he1t, logits, dst, m3)

    NB = 2000
    out = pl.pallas_call(
        _epilogue_body,
        grid=(N // NB,),
        in_specs=[pl.BlockSpec((NC, 3, NB, ROWW), lambda k: (0, 0, k, 0)),
                  pl.BlockSpec((NB, 3 * G), lambda k: (k, 0)),
                  pl.BlockSpec((3, G, G), lambda k: (0, 0, 0)),
                  pl.BlockSpec((3, G), lambda k: (0, 0)),
                  pl.BlockSpec((3 * G, G), lambda k: (0, 0)),
                  pl.BlockSpec((G,), lambda k: (0,)),
                  pl.BlockSpec((3 * G, G), lambda k: (0, 0)),
                  pl.BlockSpec((G,), lambda k: (0,)),
                  pl.BlockSpec((G, 3 * G), lambda k: (0, 0)),
                  pl.BlockSpec((3 * G,), lambda k: (0,)),
                  pl.BlockSpec((G, 3 * G), lambda k: (0, 0)),
                  pl.BlockSpec((3 * G,), lambda k: (0,))],
        out_specs=pl.BlockSpec((NB, G), lambda k: (k, 0)),
        out_shape=jax.ShapeDtypeStruct((N, G), f32),
    )(pacc, hv_cat, wet, bet, p['Wmca'], p['bmca'], p['Wmcn'], p['bmcn'],
      p['W_ih'], p['b_ih'], p['W_hh'], p['b_hh'])
    return out


# speedup vs baseline: 3.2528x; 1.2854x over previous
import functools

import numpy as np

import jax
import jax.numpy as jnp
from jax import lax
from jax.experimental import pallas as pl
from jax.experimental.pallas import tpu as pltpu
from jax.experimental.pallas import tpu_sc as plsc

N = 10000
E = 320000
DN = 128
DE = 16
G = 128

NC = 2
NS = 16
NW = NC * NS
CH = 64
NCH = 158
EPT = NCH * CH
E_PAD = NW * EPT
N_ACC = 10016
RPS = N_ACC // NS
ROWW = 144

_mesh = plsc.VectorSubcoreMesh(core_axis_name="c", subcore_axis_name="s")
_sc_params = pltpu.CompilerParams(use_tc_tiling_on_sc=False,
                                  needs_layout_passes=False)


def _lrelu(x):
    return jnp.maximum(x, 0.01 * x)


def _node_dense_body(nf_ref, wn_ref, bn_ref, we1n_ref, w2blk_ref, b2_ref,
                     hv_ref, u_ref, sn_ref):
    nf = nf_ref[...]
    hv = _lrelu(jnp.dot(nf, wn_ref[...], preferred_element_type=jnp.float32)
                + bn_ref[...][None, :])
    hv_ref[...] = hv
    u_ref[...] = jnp.dot(nf, we1n_ref[...], preferred_element_type=jnp.float32)
    sn_ref[...] = jnp.dot(hv, w2blk_ref[...],
                          preferred_element_type=jnp.float32) + b2_ref[...][None, :]


def _edge_v_body(ef_ref, we1e_ref, be1_ref, v_ref):
    v = jnp.dot(ef_ref[...], we1e_ref[...],
                preferred_element_type=jnp.float32) + be1_ref[...][None, :]
    v_ref[...] = v.astype(jnp.bfloat16)


def _max_reduce_body(mpart_ref, m_ref):
    m = jnp.max(mpart_ref[...], axis=0)
    mt = jnp.transpose(m, (1, 0))
    m_ref[...] = jnp.concatenate(
        [mt, jnp.zeros((mt.shape[0], 13), jnp.float32)], axis=1)


def _take16(x, idx):
    return x.at[idx].get(mode="promise_in_bounds")


def _sc_pass1(u_hbm, v_hbm, sn_hbm, ei_hbm, w2_hbm,
              l_hbm, t_hbm,
              ub0, ub1, vb0, vb1, eb0, eb1, ls0, ls1, sn0, sn1, ts, dotb, w2b,
              semu0, semu1, semv0, semv1, semtw, semlw0, semlw1,
              semsn0, semsn1):
    c = lax.axis_index("c")
    s = lax.axis_index("s")
    w = c * NS + s
    base0 = w * EPT
    ubufs, vbufs, ebufs, lsts = (ub0, ub1), (vb0, vb1), (eb0, eb1), (ls0, ls1)
    snbufs = (sn0, sn1)
    semus, semvs = (semu0, semu1), (semv0, semv1)
    semlws = (semlw0, semlw1)
    semsns = (semsn0, semsn1)

    pltpu.sync_copy(w2_hbm, w2b)
    w2v = [[w2b[i * 8 + j, :] for j in range(8)] for i in range(3)]

    lanes = lax.iota(jnp.int32, 16)

    def issue(b, ch):
        base = base0 + ch * CH
        pltpu.sync_copy(ei_hbm.at[:, pl.ds(base, CH)], ebufs[b])
        pltpu.async_copy(u_hbm.at[ebufs[b].at[0]], ubufs[b], semus[b])
        pltpu.async_copy(v_hbm.at[pl.ds(base, CH)], vbufs[b], semvs[b])
        pltpu.async_copy(sn_hbm.at[ebufs[b].at[1]], snbufs[b], semsns[b])

    def wait_in(b):
        pltpu.make_async_copy(u_hbm.at[pl.ds(0, CH)], ubufs[b],
                              semus[b]).wait()
        pltpu.make_async_copy(v_hbm.at[pl.ds(0, CH)], vbufs[b],
                              semvs[b]).wait()
        pltpu.make_async_copy(sn_hbm.at[pl.ds(0, CH)], snbufs[b],
                              semsns[b]).wait()

    def drain_out(b):
        pltpu.make_async_copy(lsts[b], l_hbm.at[pl.ds(0, CH)],
                              semlws[b]).wait()

    def drain_tw(ch):
        @pl.when(ch >= 1)
        def _():
            for i in range(3):
                pltpu.make_async_copy(ts.at[i], t_hbm.at[i, pl.ds(0, CH)],
                                      semtw).wait()

    def compute(b, ch):
        base = base0 + ch * CH
        ub, vb, ebf, ls, snb = ubufs[b], vbufs[b], ebufs[b], lsts[b], snbufs[b]
        drain_tw(ch)

        def g_body(g, _, ub=ub, vb=vb, ls=ls, snb=snb):
            ev = lanes + g * 16
            for i in range(3):
                ihead = jnp.full((16,), i, jnp.int32)

                def e_body(e16, _, g=g, i=i, ub=ub, vb=vb):
                    e = g * 16 + e16
                    acc = None
                    for j2 in range(4):
                        va, vb2 = plsc.unpack(
                            vb[e, pl.ds(i * G + j2 * 32, 32)],
                            format=plsc.PackFormat.INTERLEAVED)
                        u0 = ub[e, pl.ds(i * G + j2 * 32, 16)]
                        u1 = ub[e, pl.ds(i * G + j2 * 32 + 16, 16)]
                        g0 = u0 + va
                        g1 = u1 + vb2
                        t0 = jnp.maximum(g0, 0.01 * g0)
                        t1 = jnp.maximum(g1, 0.01 * g1)
                        ts[i, e, pl.ds(j2 * 32, 32)] = plsc.pack(
                            t0, t1, format=plsc.PackFormat.INTERLEAVED)
                        pj = t0 * w2v[i][2 * j2] + t1 * w2v[i][2 * j2 + 1]
                        acc = pj if acc is None else acc + pj
                    dotb[e16, :] = acc
                    return 0

                lax.fori_loop(0, 16, e_body, 0)
                se = None
                for j in range(16):
                    cj = plsc.load_gather(
                        dotb, [lanes, jnp.full((16,), j, jnp.int32)])
                    se = cj if se is None else se + cj
                snv = plsc.load_gather(snb, [ev, ihead])
                z = snv + se
                lg = jnp.maximum(z, 0.01 * z)
                plsc.store_scatter(ls, [ev, ihead], lg)
            return 0

        lax.fori_loop(0, CH // 16, g_body, 0)
        for i in range(3):
            pltpu.async_copy(ts.at[i], t_hbm.at[i, pl.ds(base, CH)], semtw)
        pltpu.async_copy(ls, l_hbm.at[pl.ds(base, CH)], semlws[b])

    issue(0, 0)

    def pair(gp, _):
        for b in (0, 1):
            ch = gp * 2 + b
            nb = 1 - b
            wait_in(b)

            @pl.when(ch + 1 < NCH)
            def _(b=b, nb=nb, ch=ch):
                @pl.when(ch >= 1)
                def _():
                    drain_out(nb)
                issue(nb, ch + 1)

            compute(b, ch)
        return 0

    lax.fori_loop(0, NCH // 2, pair, 0)
    drain_out(0)
    drain_out(1)
    for i in range(3):
        pltpu.make_async_copy(ts.at[i], t_hbm.at[i, pl.ds(0, CH)],
                              semtw).wait()


def _sc_segmax(l_hbm, dst_hbm, mpart_hbm,
               mt, lb0, lb1, db0, db1, seml0, seml1, semd0, semd1):
    c = lax.axis_index("c")
    s = lax.axis_index("s")
    w = c * NS + s
    base0 = w * EPT
    lbufs, dbufs = (lb0, lb1), (db0, db1)
    semls, semds = (seml0, seml1), (semd0, semd1)

    neg = jnp.full((16,), -1e30, jnp.float32)
    for i in range(3):
        def _init(j, _, i=i):
            mt[i, pl.ds(j * 16, 16)] = neg
            return 0
        lax.fori_loop(0, N // 16, _init, 0)

    lanes = lax.iota(jnp.int32, 16)

    def issue(b, ch):
        base = base0 + ch * CH
        pltpu.async_copy(l_hbm.at[pl.ds(base, CH)], lbufs[b], semls[b])
        pltpu.async_copy(dst_hbm.at[pl.ds(base, CH)], dbufs[b], semds[b])

    def wait_in(b):
        pltpu.make_async_copy(l_hbm.at[pl.ds(0, CH)], lbufs[b],
                              semls[b]).wait()
        pltpu.make_async_copy(dst_hbm.at[pl.ds(0, CH)], dbufs[b],
                              semds[b]).wait()

    def compute(b, ch):
        base = base0 + ch * CH
        lb, db = lbufs[b], dbufs[b]

        def g_body(g, _, lb=lb, db=db):
            ev = lanes + g * 16
            dv = db[pl.ds(g * 16, 16)]
            valid = (lanes + (base + g * 16)) < E
            for i in range(3):
                ihead = jnp.full((16,), i, jnp.int32)
                lv = plsc.load_gather(lb, [ev, ihead])
                lg_eff = jnp.where(valid, lv, -1e30)
                sk, sv = plsc.sort_key_val(dv, lg_eff)
                for sh in (1, 2, 4, 8):
                    idx = jnp.maximum(lanes - sh, 0)
                    xk = _take16(sk, idx)
                    xv = _take16(sv, idx)
                    ok = (lanes >= sh) & (xk == sk)
                    sv = jnp.where(ok, jnp.maximum(sv, xv), sv)
                nxt = _take16(sk, jnp.minimum(lanes + 1, 15))
                last = (sk != nxt) | (lanes == 15)
                cur = plsc.load_gather(mt, [ihead, sk])
                plsc.store_scatter(mt, [ihead, sk], jnp.maximum(cur, sv),
                                   mask=last)
            return 0

        lax.fori_loop(0, CH // 16, g_body, 0)

    issue(0, 0)

    def pair(gp, _):
        for b in (0, 1):
            ch = gp * 2 + b
            nb = 1 - b
            wait_in(b)

            @pl.when(ch + 1 < NCH)
            def _(b=b, nb=nb, ch=ch):
                issue(nb, ch + 1)

            compute(b, ch)
        return 0

    lax.fori_loop(0, NCH // 2, pair, 0)
    pltpu.sync_copy(mt, mpart_hbm.at[w])


def _sc_pass2(t_hbm, l_hbm, dst_hbm, m_hbm,
              pacc_hbm,
              acc, tb0, tb1, rw0, rw1, mb0, mb1, lb0, lb1, db0, db1, exb,
              semt0, semt1, seml0, seml1, semm0, semm1, sems0, sems1):
    c = lax.axis_index("c")
    s = lax.axis_index("s")
    w = c * NS + s
    base0 = w * EPT
    row0 = s * RPS
    tbufs, rowss, mbufs = (tb0, tb1), (rw0, rw1), (mb0, mb1)
    lbufs, dbufs = (lb0, lb1), (db0, db1)
    semts, semls = (semt0, semt1), (seml0, seml1)
    semms, semss = (semm0, semm1), (sems0, sems1)

    zero16 = jnp.zeros((16,), jnp.float32)
    lanes = lax.iota(jnp.int32, 16)
    colex = jnp.full((16,), G, jnp.int32)
    TAIL = RPS - (RPS // CH) * CH

    for i in range(3):
        def _zr(e, _):
            for k in range(ROWW // 16):
                rw0[e, pl.ds(k * 16, 16)] = zero16
                rw1[e, pl.ds(k * 16, 16)] = zero16
            return 0
        lax.fori_loop(0, CH, _zr, 0)
        for j in range(RPS // CH):
            pltpu.sync_copy(rw0, acc.at[pl.ds(row0 + j * CH, CH)])
        pltpu.sync_copy(rw0.at[pl.ds(0, TAIL)],
                        acc.at[pl.ds(row0 + (RPS // CH) * CH, TAIL)])
        plsc.subcore_barrier()
        ihead = jnp.full((16,), i, jnp.int32)

        def issue(b, ch, i=i):
            base = base0 + ch * CH
            pltpu.sync_copy(dst_hbm.at[pl.ds(base, CH)], dbufs[b])
            pltpu.async_copy(t_hbm.at[i, pl.ds(base, CH)], tbufs[b],
                             semts[b])
            pltpu.async_copy(l_hbm.at[pl.ds(base, CH)], lbufs[b], semls[b])
            pltpu.async_copy(m_hbm.at[dbufs[b]], mbufs[b], semms[b])

        def wait_in(b, i=i):
            pltpu.make_async_copy(t_hbm.at[i, pl.ds(0, CH)], tbufs[b],
                                  semts[b]).wait()
            pltpu.make_async_copy(l_hbm.at[pl.ds(0, CH)], lbufs[b],
                                  semls[b]).wait()
            pltpu.make_async_copy(m_hbm.at[pl.ds(0, CH)], mbufs[b],
                                  semms[b]).wait()

        def drain_sc(b):
            pltpu.make_async_copy(rowss[b], acc.at[pl.ds(0, CH)],
                                  semss[b]).wait()

        def compute(b, ch, ihead=ihead):
            base = base0 + ch * CH
            tb, rows, mb, lb = tbufs[b], rowss[b], mbufs[b], lbufs[b]

            def g_body(g, _, rows=rows, mb=mb, lb=lb):
                ev = lanes + g * 16
                mv = plsc.load_gather(mb, [ev, ihead])
                lv = plsc.load_gather(lb, [ev, ihead])
                ex = jnp.exp(lv - mv)
                mask = (lanes + (base + g * 16)) < E
                ex = jnp.where(mask, ex, 0.0)
                plsc.store_scatter(rows, [ev, colex], ex)
                exb[pl.ds(g * 16, 16)] = ex
                return 0

            lax.fori_loop(0, CH // 16, g_body, 0)

            def e_body(e, _, tb=tb, rows=rows):
                exv = plsc.load_gather(exb, [jnp.zeros((16,), jnp.int32) + e])
                for j2 in range(4):
                    ta, tb2 = plsc.unpack(tb[e, pl.ds(j2 * 32, 32)],
                                          format=plsc.PackFormat.INTERLEAVED)
                    rows[e, pl.ds(j2 * 32, 16)] = ta * exv
                    rows[e, pl.ds(j2 * 32 + 16, 16)] = tb2 * exv
                return 0

            lax.fori_loop(0, CH, e_body, 0)
            pltpu.async_copy(rows, acc.at[dbufs[b]], semss[b], add=True)

        issue(0, 0)

        def pair(gp, _):
            for b in (0, 1):
                ch = gp * 2 + b
                nb = 1 - b
                wait_in(b)

                @pl.when(ch + 1 < NCH)
                def _(b=b, nb=nb, ch=ch):
                    @pl.when(ch >= 1)
                    def _():
                        drain_sc(nb)
                    issue(nb, ch + 1)

                compute(b, ch)
            return 0

        lax.fori_loop(0, NCH // 2, pair, 0)
        drain_sc(0)
        drain_sc(1)
        plsc.subcore_barrier()

        @pl.when(s < NS - 1)
        def _():
            pltpu.sync_copy(acc.at[pl.ds(row0, RPS)],
                            pacc_hbm.at[c, i, pl.ds(row0, RPS)])

        @pl.when(s == NS - 1)
        def _():
            pltpu.sync_copy(acc.at[pl.ds(row0, N - (NS - 1) * RPS)],
                            pacc_hbm.at[c, i, pl.ds(row0, N - (NS - 1) * RPS)])
        plsc.subcore_barrier()


def _epilogue_body(pacc_ref, hv_ref,
                   wet_ref, bet_ref, wmca_ref, bmca_ref, wmcn_ref, bmcn_ref,
                   wih_ref, bih_ref, whh_ref, bhh_ref,
                   out_ref):
    pacc = pacc_ref[...]
    hv = hv_ref[...]
    ctx = []
    for i in range(3):
        p = pacc[0, i] + pacc[1, i]
        pi = p[:, :G]
        si = p[:, G]
        re = jnp.where(si > 0, 1.0 / jnp.where(si > 0, si, 1.0), 0.0)
        a = pi * re[:, None]
        ci = jnp.dot(a, wet_ref[...][i], preferred_element_type=jnp.float32)
        ci = ci + jnp.where(si > 0, 1.0, 0.0)[:, None] * bet_ref[...][i][None, :]
        ctx.append(jnp.where(ci > 0, ci, jnp.exp(jnp.minimum(ci, 0.0)) - 1.0))
    context = jnp.dot(jnp.concatenate(ctx, axis=1), wmca_ref[...],
                      preferred_element_type=jnp.float32) + bmca_ref[...][None, :]
    hnode = jnp.dot(hv, wmcn_ref[...],
                    preferred_element_type=jnp.float32) + bmcn_ref[...][None, :]
    gi = jnp.dot(context, wih_ref[...],
                 preferred_element_type=jnp.float32) + bih_ref[...][None, :]
    gh = jnp.dot(hnode, whh_ref[...],
                 preferred_element_type=jnp.float32) + bhh_ref[...][None, :]
    i_r, i_z, i_n = gi[:, :G], gi[:, G:2 * G], gi[:, 2 * G:]
    h_r, h_z, h_n = gh[:, :G], gh[:, G:2 * G], gh[:, 2 * G:]
    r = jax.nn.sigmoid(i_r + h_r)
    z = jax.nn.sigmoid(i_z + h_z)
    cand = jnp.tanh(i_n + r * h_n)
    h_new = (1.0 - z) * cand + z * hnode
    out_ref[...] = jnp.maximum(h_new, 0.0)


def kernel(node_feats, edge_feats, params, edge_index):
    p = params
    f32 = jnp.float32

    wn_cat = jnp.concatenate([p['Wn%d' % i] for i in (1, 2, 3)], axis=1)
    bn_cat = jnp.concatenate([p['bn%d' % i] for i in (1, 2, 3)], axis=0)
    we1n_cat = jnp.concatenate([p['We1_%d' % i][:DN] for i in (1, 2, 3)], axis=1)
    we1e_cat = jnp.concatenate([p['We1_%d' % i][DN:] for i in (1, 2, 3)], axis=1)
    be1_cat = jnp.concatenate([p['be1_%d' % i] for i in (1, 2, 3)], axis=0)
    pidx = np.empty(3 * G, np.int32)
    for cblk in range(3 * G // 32):
        pidx[cblk * 32 + 2 * np.arange(16)] = cblk * 32 + np.arange(16)
        pidx[cblk * 32 + 2 * np.arange(16) + 1] = cblk * 32 + 16 + np.arange(16)
    we1e_cat = we1e_cat[:, pidx]
    be1_cat = be1_cat[pidx]
    w2blk = jnp.zeros((3 * G, 16), f32)
    for i in (1, 2, 3):
        w2blk = w2blk.at[(i - 1) * G:i * G, i - 1].set(p['We2_%d' % i][:G, 0])
    b2 = jnp.zeros((16,), f32)
    for i in (1, 2, 3):
        b2 = b2.at[i - 1].set(p['be2_%d' % i][0])
    w2b = jnp.stack([p['We2_%d' % i][G:, 0] for i in (1, 2, 3)], axis=0)
    wet = jnp.stack([p['Wet%d' % i] for i in (1, 2, 3)], axis=0)
    bet = jnp.stack([p['bet%d' % i] for i in (1, 2, 3)], axis=0)

    ei_pad = jnp.pad(edge_index.astype(jnp.int32), ((0, 0), (0, E_PAD - E)))
    dst = ei_pad[1]
    ef_pad = jnp.pad(edge_feats, ((0, E_PAD - E), (0, 0)))

    hv_cat, u_cat, sn3 = pl.pallas_call(
        _node_dense_body,
        out_shape=[jax.ShapeDtypeStruct((N, 3 * G), f32),
                   jax.ShapeDtypeStruct((N, 3 * G), f32),
                   jax.ShapeDtypeStruct((N, 16), f32)],
    )(node_feats, wn_cat, bn_cat, we1n_cat, w2blk, b2)

    EB = 2528
    v_cat = pl.pallas_call(
        _edge_v_body,
        grid=(E_PAD // EB,),
        in_specs=[pl.BlockSpec((EB, DE), lambda i: (i, 0)),
                  pl.BlockSpec((DE, 3 * G), lambda i: (0, 0)),
                  pl.BlockSpec((3 * G,), lambda i: (0,))],
        out_specs=pl.BlockSpec((EB, 3 * G), lambda i: (i, 0)),
        out_shape=jax.ShapeDtypeStruct((E_PAD, 3 * G), jnp.bfloat16),
    )(ef_pad, we1e_cat, be1_cat)

    pass1 = pl.kernel(
        _sc_pass1,
        out_type=[jax.ShapeDtypeStruct((E_PAD, 4), f32),
                  jax.ShapeDtypeStruct((3, E_PAD, G), jnp.bfloat16)],
        mesh=_mesh,
        scratch_types=[
            pltpu.VMEM((CH, 3 * G), f32),
            pltpu.VMEM((CH, 3 * G), f32),
            pltpu.VMEM((CH, 3 * G), jnp.bfloat16),
            pltpu.VMEM((CH, 3 * G), jnp.bfloat16),
            pltpu.VMEM((2, CH), jnp.int32),
            pltpu.VMEM((2, CH), jnp.int32),
            pltpu.VMEM((CH, 4), f32),
            pltpu.VMEM((CH, 4), f32),
            pltpu.VMEM((CH, 16), f32),
            pltpu.VMEM((CH, 16), f32),
            pltpu.VMEM((3, CH, G), jnp.bfloat16),
            pltpu.VMEM((16, 16), f32),
            pltpu.VMEM((24, 16), f32),
        ] + [pltpu.SemaphoreType.DMA] * 9,
        compiler_params=_sc_params,
    )
    logits, he1t = pass1(u_cat, v_cat, sn3, ei_pad, w2b.reshape(24, 16))

    segmax = pl.kernel(
        _sc_segmax,
        out_type=jax.ShapeDtypeStruct((NW, 3, N), f32),
        mesh=_mesh,
        scratch_types=[
            pltpu.VMEM((3, N), f32),
            pltpu.VMEM((CH, 4), f32),
            pltpu.VMEM((CH, 4), f32),
            pltpu.VMEM((CH,), jnp.int32),
            pltpu.VMEM((CH,), jnp.int32),
        ] + [pltpu.SemaphoreType.DMA] * 4,
        compiler_params=_sc_params,
    )
    mpart = segmax(logits, dst)

    m3 = pl.pallas_call(
        _max_reduce_body,
        out_shape=jax.ShapeDtypeStruct((N, 16), f32),
    )(mpart)

    pass2 = pl.kernel(
        _sc_pass2,
        out_type=jax.ShapeDtypeStruct((NC, 3, N, ROWW), f32),
        mesh=_mesh,
        scratch_types=[
            pltpu.VMEM_SHARED((N_ACC, ROWW), f32),
            pltpu.VMEM((CH, G), jnp.bfloat16),
            pltpu.VMEM((CH, G), jnp.bfloat16),
            pltpu.VMEM((CH, ROWW), f32),
            pltpu.VMEM((CH, ROWW), f32),
            pltpu.VMEM((CH, 16), f32),
            pltpu.VMEM((CH, 16), f32),
            pltpu.VMEM((CH, 4), f32),
            pltpu.VMEM((CH, 4), f32),
            pltpu.VMEM((CH,), jnp.int32),
            pltpu.VMEM((CH,), jnp.int32),
            pltpu.VMEM((CH,), f32),
        ] + [pltpu.SemaphoreType.DMA] * 8,
        compiler_params=_sc_params,
    )
    pacc = pass2(he1t, logits, dst, m3)

    NB = 2000
    out = pl.pallas_call(
        _epilogue_body,
        grid=(N // NB,),
        in_specs=[pl.BlockSpec((NC, 3, NB, ROWW), lambda k: (0, 0, k, 0)),
                  pl.BlockSpec((NB, 3 * G), lambda k: (k, 0)),
                  pl.BlockSpec((3, G, G), lambda k: (0, 0, 0)),
                  pl.BlockSpec((3, G), lambda k: (0, 0)),
                  pl.BlockSpec((3 * G, G), lambda k: (0, 0)),
                  pl.BlockSpec((G,), lambda k: (0,)),
                  pl.BlockSpec((3 * G, G), lambda k: (0, 0)),
                  pl.BlockSpec((G,), lambda k: (0,)),
                  pl.BlockSpec((G, 3 * G), lambda k: (0, 0)),
                  pl.BlockSpec((3 * G,), lambda k: (0,)),
                  pl.BlockSpec((G, 3 * G), lambda k: (0, 0)),
                  pl.BlockSpec((3 * G,), lambda k: (0,))],
        out_specs=pl.BlockSpec((NB, G), lambda k: (k, 0)),
        out_shape=jax.ShapeDtypeStruct((N, G), f32),
    )(pacc, hv_cat, wet, bet, p['Wmca'], p['bmca'], p['Wmcn'], p['bmcn'],
      p['W_ih'], p['b_ih'], p['W_hh'], p['b_hh'])
    return out


# speedup vs baseline: 3.2603x; 1.0023x over previous
import functools

import numpy as np

import jax
import jax.numpy as jnp
from jax import lax
from jax.experimental import pallas as pl
from jax.experimental.pallas import tpu as pltpu
from jax.experimental.pallas import tpu_sc as plsc

N = 10000
E = 320000
DN = 128
DE = 16
G = 128

NC = 2
NS = 16
NW = NC * NS
CH = 64
NCH = 158
EPT = NCH * CH
E_PAD = NW * EPT
N_ACC = 10016
RPS = N_ACC // NS
ROWW = 144

_mesh = plsc.VectorSubcoreMesh(core_axis_name="c", subcore_axis_name="s")
_sc_params = pltpu.CompilerParams(use_tc_tiling_on_sc=False,
                                  needs_layout_passes=False)


def _lrelu(x):
    return jnp.maximum(x, 0.01 * x)


def _node_dense_body(nf_ref, wn_ref, bn_ref, we1n_ref, w2blk_ref, b2_ref,
                     hv_ref, u_ref, sn_ref):
    nf = nf_ref[...]
    hv = _lrelu(jnp.dot(nf, wn_ref[...], preferred_element_type=jnp.float32)
                + bn_ref[...][None, :])
    hv_ref[...] = hv
    u_ref[...] = jnp.dot(nf, we1n_ref[...], preferred_element_type=jnp.float32)
    sn_ref[...] = jnp.dot(hv, w2blk_ref[...],
                          preferred_element_type=jnp.float32) + b2_ref[...][None, :]


def _edge_v_body(ef_ref, we1e_ref, be1_ref, v_ref):
    v = jnp.dot(ef_ref[...], we1e_ref[...],
                preferred_element_type=jnp.float32) + be1_ref[...][None, :]
    v_ref[...] = v.astype(jnp.bfloat16)


def _max_reduce_body(mpart_ref, m_ref):
    m = jnp.max(mpart_ref[...], axis=0)
    mt = jnp.transpose(m, (1, 0))
    m_ref[...] = jnp.concatenate(
        [mt, jnp.zeros((mt.shape[0], 13), jnp.float32)], axis=1)


def _take16(x, idx):
    return x.at[idx].get(mode="promise_in_bounds")


def _sc_pass1(u_hbm, v_hbm, sn_hbm, ei_hbm, w2_hbm,
              l_hbm, t_hbm,
              ub0, ub1, vb0, vb1, eb0, eb1, ls0, ls1, sn0, sn1, ts, dotb, w2b,
              semu0, semu1, semv0, semv1, semtw, semlw0, semlw1,
              semsn0, semsn1):
    c = lax.axis_index("c")
    s = lax.axis_index("s")
    w = c * NS + s
    base0 = w * EPT
    ubufs, vbufs, ebufs, lsts = (ub0, ub1), (vb0, vb1), (eb0, eb1), (ls0, ls1)
    snbufs = (sn0, sn1)
    semus, semvs = (semu0, semu1), (semv0, semv1)
    semlws = (semlw0, semlw1)
    semsns = (semsn0, semsn1)

    pltpu.sync_copy(w2_hbm, w2b)
    w2v = [[w2b[i * 8 + j, :] for j in range(8)] for i in range(3)]

    lanes = lax.iota(jnp.int32, 16)

    def issue(b, ch):
        base = base0 + ch * CH
        pltpu.sync_copy(ei_hbm.at[:, pl.ds(base, CH)], ebufs[b])
        pltpu.async_copy(u_hbm.at[ebufs[b].at[0]], ubufs[b], semus[b])
        pltpu.async_copy(v_hbm.at[pl.ds(base, CH)], vbufs[b], semvs[b])
        pltpu.async_copy(sn_hbm.at[ebufs[b].at[1]], snbufs[b], semsns[b])

    def wait_in(b):
        pltpu.make_async_copy(u_hbm.at[pl.ds(0, CH)], ubufs[b],
                              semus[b]).wait()
        pltpu.make_async_copy(v_hbm.at[pl.ds(0, CH)], vbufs[b],
                              semvs[b]).wait()
        pltpu.make_async_copy(sn_hbm.at[pl.ds(0, CH)], snbufs[b],
                              semsns[b]).wait()

    def drain_out(b):
        pltpu.make_async_copy(lsts[b], l_hbm.at[pl.ds(0, CH * 4)],
                              semlws[b]).wait()

    def drain_tw(ch):
        @pl.when(ch >= 1)
        def _():
            for i in range(3):
                pltpu.make_async_copy(ts.at[i], t_hbm.at[pl.ds(0, CH * G)],
                                      semtw).wait()

    def compute(b, ch):
        base = base0 + ch * CH
        ub, vb, ebf, ls, snb = ubufs[b], vbufs[b], ebufs[b], lsts[b], snbufs[b]
        drain_tw(ch)

        def g_body(g, _, ub=ub, vb=vb, ls=ls, snb=snb):
            ev = lanes + g * 16
            for i in range(3):
                ihead = jnp.full((16,), i, jnp.int32)

                def e_body(e16, _, g=g, i=i, ub=ub, vb=vb):
                    e = g * 16 + e16
                    acc = None
                    for j2 in range(4):
                        va, vb2 = plsc.unpack(
                            vb[e, pl.ds(i * G + j2 * 32, 32)],
                            format=plsc.PackFormat.INTERLEAVED)
                        u0 = ub[e, pl.ds(i * G + j2 * 32, 16)]
                        u1 = ub[e, pl.ds(i * G + j2 * 32 + 16, 16)]
                        g0 = u0 + va
                        g1 = u1 + vb2
                        t0 = jnp.maximum(g0, 0.01 * g0)
                        t1 = jnp.maximum(g1, 0.01 * g1)
                        ts[i, pl.ds(e * G + j2 * 32, 32)] = plsc.pack(
                            t0, t1, format=plsc.PackFormat.INTERLEAVED)
                        pj = t0 * w2v[i][2 * j2] + t1 * w2v[i][2 * j2 + 1]
                        acc = pj if acc is None else acc + pj
                    dotb[e16, :] = acc
                    return 0

                lax.fori_loop(0, 16, e_body, 0)
                se = None
                for j in range(16):
                    cj = plsc.load_gather(
                        dotb, [lanes, jnp.full((16,), j, jnp.int32)])
                    se = cj if se is None else se + cj
                snv = plsc.load_gather(snb, [ev, ihead])
                z = snv + se
                lg = jnp.maximum(z, 0.01 * z)
                plsc.store_scatter(ls, [ev * 4 + ihead], lg)
            return 0

        lax.fori_loop(0, CH // 16, g_body, 0)
        for i in range(3):
            pltpu.async_copy(ts.at[i],
                             t_hbm.at[pl.ds(i * (E_PAD * G) + base * G,
                                            CH * G)], semtw)
        pltpu.async_copy(ls, l_hbm.at[pl.ds(base * 4, CH * 4)], semlws[b])

    issue(0, 0)

    def pair(gp, _):
        for b in (0, 1):
            ch = gp * 2 + b
            nb = 1 - b
            wait_in(b)

            @pl.when(ch + 1 < NCH)
            def _(b=b, nb=nb, ch=ch):
                @pl.when(ch >= 1)
                def _():
                    drain_out(nb)
                issue(nb, ch + 1)

            compute(b, ch)
        return 0

    lax.fori_loop(0, NCH // 2, pair, 0)
    drain_out(0)
    drain_out(1)
    for i in range(3):
        pltpu.make_async_copy(ts.at[i], t_hbm.at[pl.ds(0, CH * G)],
                              semtw).wait()


def _sc_segmax(l_hbm, dst_hbm, mpart_hbm,
               mt, lb0, lb1, db0, db1, seml0, seml1, semd0, semd1):
    c = lax.axis_index("c")
    s = lax.axis_index("s")
    w = c * NS + s
    base0 = w * EPT
    lbufs, dbufs = (lb0, lb1), (db0, db1)
    semls, semds = (seml0, seml1), (semd0, semd1)

    neg = jnp.full((16,), -1e30, jnp.float32)
    for i in range(3):
        def _init(j, _, i=i):
            mt[i, pl.ds(j * 16, 16)] = neg
            return 0
        lax.fori_loop(0, N // 16, _init, 0)

    lanes = lax.iota(jnp.int32, 16)

    def issue(b, ch):
        base = base0 + ch * CH
        pltpu.async_copy(l_hbm.at[pl.ds(base * 4, CH * 4)], lbufs[b],
                         semls[b])
        pltpu.async_copy(dst_hbm.at[pl.ds(base, CH)], dbufs[b], semds[b])

    def wait_in(b):
        pltpu.make_async_copy(l_hbm.at[pl.ds(0, CH * 4)], lbufs[b],
                              semls[b]).wait()
        pltpu.make_async_copy(dst_hbm.at[pl.ds(0, CH)], dbufs[b],
                              semds[b]).wait()

    def compute(b, ch):
        base = base0 + ch * CH
        lb, db = lbufs[b], dbufs[b]

        def g_body(g, _, lb=lb, db=db):
            ev = lanes + g * 16
            dv = db[pl.ds(g * 16, 16)]
            valid = (lanes + (base + g * 16)) < E
            for i in range(3):
                ihead = jnp.full((16,), i, jnp.int32)
                lv = plsc.load_gather(lb, [ev * 4 + ihead])
                lg_eff = jnp.where(valid, lv, -1e30)
                sk, sv = plsc.sort_key_val(dv, lg_eff)
                for sh in (1, 2, 4, 8):
                    idx = jnp.maximum(lanes - sh, 0)
                    xk = _take16(sk, idx)
                    xv = _take16(sv, idx)
                    ok = (lanes >= sh) & (xk == sk)
                    sv = jnp.where(ok, jnp.maximum(sv, xv), sv)
                nxt = _take16(sk, jnp.minimum(lanes + 1, 15))
                last = (sk != nxt) | (lanes == 15)
                cur = plsc.load_gather(mt, [ihead, sk])
                plsc.store_scatter(mt, [ihead, sk], jnp.maximum(cur, sv),
                                   mask=last)
            return 0

        lax.fori_loop(0, CH // 16, g_body, 0)

    issue(0, 0)

    def pair(gp, _):
        for b in (0, 1):
            ch = gp * 2 + b
            nb = 1 - b
            wait_in(b)

            @pl.when(ch + 1 < NCH)
            def _(b=b, nb=nb, ch=ch):
                issue(nb, ch + 1)

            compute(b, ch)
        return 0

    lax.fori_loop(0, NCH // 2, pair, 0)
    pltpu.sync_copy(mt, mpart_hbm.at[w])


def _sc_pass2(t_hbm, l_hbm, dst_hbm, m_hbm,
              pacc_hbm,
              acc, tb0, tb1, rw0, rw1, mb0, mb1, lb0, lb1, db0, db1, exb,
              semt0, semt1, seml0, seml1, semm0, semm1, sems0, sems1):
    c = lax.axis_index("c")
    s = lax.axis_index("s")
    w = c * NS + s
    base0 = w * EPT
    row0 = s * RPS
    tbufs, rowss, mbufs = (tb0, tb1), (rw0, rw1), (mb0, mb1)
    lbufs, dbufs = (lb0, lb1), (db0, db1)
    semts, semls = (semt0, semt1), (seml0, seml1)
    semms, semss = (semm0, semm1), (sems0, sems1)

    zero16 = jnp.zeros((16,), jnp.float32)
    lanes = lax.iota(jnp.int32, 16)
    colex = jnp.full((16,), G, jnp.int32)
    TAIL = RPS - (RPS // CH) * CH

    for i in range(3):
        def _zr(e, _):
            for k in range(ROWW // 16):
                rw0[e, pl.ds(k * 16, 16)] = zero16
                rw1[e, pl.ds(k * 16, 16)] = zero16
            return 0
        lax.fori_loop(0, CH, _zr, 0)
        for j in range(RPS // CH):
            pltpu.sync_copy(rw0, acc.at[pl.ds(row0 + j * CH, CH)])
        pltpu.sync_copy(rw0.at[pl.ds(0, TAIL)],
                        acc.at[pl.ds(row0 + (RPS // CH) * CH, TAIL)])
        plsc.subcore_barrier()
        ihead = jnp.full((16,), i, jnp.int32)

        def issue(b, ch, i=i):
            base = base0 + ch * CH
            pltpu.sync_copy(dst_hbm.at[pl.ds(base, CH)], dbufs[b])
            pltpu.async_copy(t_hbm.at[pl.ds(i * (E_PAD * G) + base * G,
                                            CH * G)], tbufs[b], semts[b])
            pltpu.async_copy(l_hbm.at[pl.ds(base * 4, CH * 4)], lbufs[b],
                             semls[b])
            pltpu.async_copy(m_hbm.at[dbufs[b]], mbufs[b], semms[b])

        def wait_in(b, i=i):
            pltpu.make_async_copy(t_hbm.at[pl.ds(0, CH * G)], tbufs[b],
                                  semts[b]).wait()
            pltpu.make_async_copy(l_hbm.at[pl.ds(0, CH * 4)], lbufs[b],
                                  semls[b]).wait()
            pltpu.make_async_copy(m_hbm.at[pl.ds(0, CH)], mbufs[b],
                                  semms[b]).wait()

        def drain_sc(b):
            pltpu.make_async_copy(rowss[b], acc.at[pl.ds(0, CH)],
                                  semss[b]).wait()

        def compute(b, ch, ihead=ihead):
            base = base0 + ch * CH
            tb, rows, mb, lb = tbufs[b], rowss[b], mbufs[b], lbufs[b]

            def g_body(g, _, rows=rows, mb=mb, lb=lb):
                ev = lanes + g * 16
                mv = plsc.load_gather(mb, [ev, ihead])
                lv = plsc.load_gather(lb, [ev * 4 + ihead])
                ex = jnp.exp(lv - mv)
                mask = (lanes + (base + g * 16)) < E
                ex = jnp.where(mask, ex, 0.0)
                plsc.store_scatter(rows, [ev, colex], ex)
                exb[pl.ds(g * 16, 16)] = ex
                return 0

            lax.fori_loop(0, CH // 16, g_body, 0)

            def e_body(e, _, tb=tb, rows=rows):
                exv = plsc.load_gather(exb, [jnp.zeros((16,), jnp.int32) + e])
                for j2 in range(4):
                    ta, tb2 = plsc.unpack(tb[pl.ds(e * G + j2 * 32, 32)],
                                          format=plsc.PackFormat.INTERLEAVED)
                    rows[e, pl.ds(j2 * 32, 16)] = ta * exv
                    rows[e, pl.ds(j2 * 32 + 16, 16)] = tb2 * exv
                return 0

            lax.fori_loop(0, CH, e_body, 0)
            pltpu.async_copy(rows, acc.at[dbufs[b]], semss[b], add=True)

        issue(0, 0)

        def pair(gp, _):
            for b in (0, 1):
                ch = gp * 2 + b
                nb = 1 - b
                wait_in(b)

                @pl.when(ch + 1 < NCH)
                def _(b=b, nb=nb, ch=ch):
                    @pl.when(ch >= 1)
                    def _():
                        drain_sc(nb)
                    issue(nb, ch + 1)

                compute(b, ch)
            return 0

        lax.fori_loop(0, NCH // 2, pair, 0)
        drain_sc(0)
        drain_sc(1)
        plsc.subcore_barrier()

        @pl.when(s < NS - 1)
        def _():
            pltpu.sync_copy(acc.at[pl.ds(row0, RPS)],
                            pacc_hbm.at[c, i, pl.ds(row0, RPS)])

        @pl.when(s == NS - 1)
        def _():
            pltpu.sync_copy(acc.at[pl.ds(row0, N - (NS - 1) * RPS)],
                            pacc_hbm.at[c, i, pl.ds(row0, N - (NS - 1) * RPS)])
        plsc.subcore_barrier()


def _epilogue_body(pacc_ref, hv_ref,
                   wet_ref, bet_ref, wmca_ref, bmca_ref, wmcn_ref, bmcn_ref,
                   wih_ref, bih_ref, whh_ref, bhh_ref,
                   out_ref):
    pacc = pacc_ref[...]
    hv = hv_ref[...]
    ctx = []
    for i in range(3):
        p = pacc[0, i] + pacc[1, i]
        pi = p[:, :G]
        si = p[:, G]
        re = jnp.where(si > 0, 1.0 / jnp.where(si > 0, si, 1.0), 0.0)
        a = pi * re[:, None]
        ci = jnp.dot(a, wet_ref[...][i], preferred_element_type=jnp.float32)
        ci = ci + jnp.where(si > 0, 1.0, 0.0)[:, None] * bet_ref[...][i][None, :]
        ctx.append(jnp.where(ci > 0, ci, jnp.exp(jnp.minimum(ci, 0.0)) - 1.0))
    context = jnp.dot(jnp.concatenate(ctx, axis=1), wmca_ref[...],
                      preferred_element_type=jnp.float32) + bmca_ref[...][None, :]
    hnode = jnp.dot(hv, wmcn_ref[...],
                    preferred_element_type=jnp.float32) + bmcn_ref[...][None, :]
    gi = jnp.dot(context, wih_ref[...],
                 preferred_element_type=jnp.float32) + bih_ref[...][None, :]
    gh = jnp.dot(hnode, whh_ref[...],
                 preferred_element_type=jnp.float32) + bhh_ref[...][None, :]
    i_r, i_z, i_n = gi[:, :G], gi[:, G:2 * G], gi[:, 2 * G:]
    h_r, h_z, h_n = gh[:, :G], gh[:, G:2 * G], gh[:, 2 * G:]
    r = jax.nn.sigmoid(i_r + h_r)
    z = jax.nn.sigmoid(i_z + h_z)
    cand = jnp.tanh(i_n + r * h_n)
    h_new = (1.0 - z) * cand + z * hnode
    out_ref[...] = jnp.maximum(h_new, 0.0)


def kernel(node_feats, edge_feats, params, edge_index):
    p = params
    f32 = jnp.float32

    wn_cat = jnp.concatenate([p['Wn%d' % i] for i in (1, 2, 3)], axis=1)
    bn_cat = jnp.concatenate([p['bn%d' % i] for i in (1, 2, 3)], axis=0)
    we1n_cat = jnp.concatenate([p['We1_%d' % i][:DN] for i in (1, 2, 3)], axis=1)
    we1e_cat = jnp.concatenate([p['We1_%d' % i][DN:] for i in (1, 2, 3)], axis=1)
    be1_cat = jnp.concatenate([p['be1_%d' % i] for i in (1, 2, 3)], axis=0)
    pidx = np.empty(3 * G, np.int32)
    for cblk in range(3 * G // 32):
        pidx[cblk * 32 + 2 * np.arange(16)] = cblk * 32 + np.arange(16)
        pidx[cblk * 32 + 2 * np.arange(16) + 1] = cblk * 32 + 16 + np.arange(16)
    we1e_cat = we1e_cat[:, pidx]
    be1_cat = be1_cat[pidx]
    w2blk = jnp.zeros((3 * G, 16), f32)
    for i in (1, 2, 3):
        w2blk = w2blk.at[(i - 1) * G:i * G, i - 1].set(p['We2_%d' % i][:G, 0])
    b2 = jnp.zeros((16,), f32)
    for i in (1, 2, 3):
        b2 = b2.at[i - 1].set(p['be2_%d' % i][0])
    w2b = jnp.stack([p['We2_%d' % i][G:, 0] for i in (1, 2, 3)], axis=0)
    wet = jnp.stack([p['Wet%d' % i] for i in (1, 2, 3)], axis=0)
    bet = jnp.stack([p['bet%d' % i] for i in (1, 2, 3)], axis=0)

    ei_pad = jnp.pad(edge_index.astype(jnp.int32), ((0, 0), (0, E_PAD - E)))
    dst = ei_pad[1]
    ef_pad = jnp.pad(edge_feats, ((0, E_PAD - E), (0, 0)))

    hv_cat, u_cat, sn3 = pl.pallas_call(
        _node_dense_body,
        out_shape=[jax.ShapeDtypeStruct((N, 3 * G), f32),
                   jax.ShapeDtypeStruct((N, 3 * G), f32),
                   jax.ShapeDtypeStruct((N, 16), f32)],
    )(node_feats, wn_cat, bn_cat, we1n_cat, w2blk, b2)

    EB = 2528
    v_cat = pl.pallas_call(
        _edge_v_body,
        grid=(E_PAD // EB,),
        in_specs=[pl.BlockSpec((EB, DE), lambda i: (i, 0)),
                  pl.BlockSpec((DE, 3 * G), lambda i: (0, 0)),
                  pl.BlockSpec((3 * G,), lambda i: (0,))],
        out_specs=pl.BlockSpec((EB, 3 * G), lambda i: (i, 0)),
        out_shape=jax.ShapeDtypeStruct((E_PAD, 3 * G), jnp.bfloat16),
    )(ef_pad, we1e_cat, be1_cat)

    pass1 = pl.kernel(
        _sc_pass1,
        out_type=[jax.ShapeDtypeStruct((E_PAD * 4,), f32),
                  jax.ShapeDtypeStruct((3 * E_PAD * G,), jnp.bfloat16)],
        mesh=_mesh,
        scratch_types=[
            pltpu.VMEM((CH, 3 * G), f32),
            pltpu.VMEM((CH, 3 * G), f32),
            pltpu.VMEM((CH, 3 * G), jnp.bfloat16),
            pltpu.VMEM((CH, 3 * G), jnp.bfloat16),
            pltpu.VMEM((2, CH), jnp.int32),
            pltpu.VMEM((2, CH), jnp.int32),
            pltpu.VMEM((CH * 4,), f32),
            pltpu.VMEM((CH * 4,), f32),
            pltpu.VMEM((CH, 16), f32),
            pltpu.VMEM((CH, 16), f32),
            pltpu.VMEM((3, CH * G), jnp.bfloat16),
            pltpu.VMEM((16, 16), f32),
            pltpu.VMEM((24, 16), f32),
        ] + [pltpu.SemaphoreType.DMA] * 9,
        compiler_params=_sc_params,
    )
    logits, he1t = pass1(u_cat, v_cat, sn3, ei_pad, w2b.reshape(24, 16))

    segmax = pl.kernel(
        _sc_segmax,
        out_type=jax.ShapeDtypeStruct((NW, 3, N), f32),
        mesh=_mesh,
        scratch_types=[
            pltpu.VMEM((3, N), f32),
            pltpu.VMEM((CH * 4,), f32),
            pltpu.VMEM((CH * 4,), f32),
            pltpu.VMEM((CH,), jnp.int32),
            pltpu.VMEM((CH,), jnp.int32),
        ] + [pltpu.SemaphoreType.DMA] * 4,
        compiler_params=_sc_params,
    )
    mpart = segmax(logits, dst)

    m3 = pl.pallas_call(
        _max_reduce_body,
        out_shape=jax.ShapeDtypeStruct((N, 16), f32),
    )(mpart)

    pass2 = pl.kernel(
        _sc_pass2,
        out_type=jax.ShapeDtypeStruct((NC, 3, N, ROWW), f32),
        mesh=_mesh,
        scratch_types=[
            pltpu.VMEM_SHARED((N_ACC, ROWW), f32),
            pltpu.VMEM((CH * G,), jnp.bfloat16),
            pltpu.VMEM((CH * G,), jnp.bfloat16),
            pltpu.VMEM((CH, ROWW), f32),
            pltpu.VMEM((CH, ROWW), f32),
            pltpu.VMEM((CH, 16), f32),
            pltpu.VMEM((CH, 16), f32),
            pltpu.VMEM((CH * 4,), f32),
            pltpu.VMEM((CH * 4,), f32),
            pltpu.VMEM((CH,), jnp.int32),
            pltpu.VMEM((CH,), jnp.int32),
            pltpu.VMEM((CH,), f32),
        ] + [pltpu.SemaphoreType.DMA] * 8,
        compiler_params=_sc_params,
    )
    pacc = pass2(he1t, logits, dst, m3)

    NB = 2000
    out = pl.pallas_call(
        _epilogue_body,
        grid=(N // NB,),
        in_specs=[pl.BlockSpec((NC, 3, NB, ROWW), lambda k: (0, 0, k, 0)),
                  pl.BlockSpec((NB, 3 * G), lambda k: (k, 0)),
                  pl.BlockSpec((3, G, G), lambda k: (0, 0, 0)),
                  pl.BlockSpec((3, G), lambda k: (0, 0)),
                  pl.BlockSpec((3 * G, G), lambda k: (0, 0)),
                  pl.BlockSpec((G,), lambda k: (0,)),
                  pl.BlockSpec((3 * G, G), lambda k: (0, 0)),
                  pl.BlockSpec((G,), lambda k: (0,)),
                  pl.BlockSpec((G, 3 * G), lambda k: (0, 0)),
                  pl.BlockSpec((3 * G,), lambda k: (0,)),
                  pl.BlockSpec((G, 3 * G), lambda k: (0, 0)),
                  pl.BlockSpec((3 * G,), lambda k: (0,))],
        out_specs=pl.BlockSpec((NB, G), lambda k: (k, 0)),
        out_shape=jax.ShapeDtypeStruct((N, G), f32),
    )(pacc, hv_cat, wet, bet, p['Wmca'], p['bmca'], p['Wmcn'], p['bmcn'],
      p['W_ih'], p['b_ih'], p['W_hh'], p['b_hh'])
    return out
